# Initial kernel scaffold; baseline (speedup 1.0000x reference)
import jax
import jax.numpy as jnp
from jax.experimental import pallas as pl


def kernel(x, edge_index, edge_attr, w_s, w_n):
    raise NotImplementedError("write your pallas kernel here")


# speedup vs baseline: 5.3827x; 5.3827x over previous
import functools

import jax
import jax.numpy as jnp
from jax import lax
from jax.experimental import pallas as pl
from jax.experimental.pallas import tpu as pltpu
from jax.experimental.pallas import tpu_sc as plsc

N_NODES = 10000
N_EDGES = 320000
F = 128
B = 16
NC = 2
NS = 16
EPT = N_EDGES // (NC * NS)
CHUNK = 40
NCHUNK = EPT // CHUNK
ZROWS = 2 * CHUNK
WRPT = 624


def _sc_segment_sums(x, src, nbr, edge_attr):
    mesh = plsc.VectorSubcoreMesh(core_axis_name="c", subcore_axis_name="s")

    @functools.partial(
        pl.kernel,
        out_type=(
            jax.ShapeDtypeStruct((NC, N_NODES, F), jnp.float32),
            jax.ShapeDtypeStruct((NC, N_NODES, B), jnp.float32),
            jax.ShapeDtypeStruct((NC, N_NODES, B), jnp.float32),
        ),
        mesh=mesh,
        compiler_params=pltpu.CompilerParams(use_tc_tiling_on_sc=False),
        scratch_types=(
            pltpu.VMEM_SHARED((N_NODES, F), jnp.float32),
            pltpu.VMEM_SHARED((N_NODES, B), jnp.float32),
            pltpu.VMEM_SHARED((N_NODES, B), jnp.float32),
            pltpu.VMEM((2, CHUNK), jnp.int32),
            pltpu.VMEM((2, CHUNK), jnp.int32),
            pltpu.VMEM((2 * CHUNK, F), jnp.float32),
            pltpu.VMEM((2 * CHUNK, B), jnp.float32),
            pltpu.VMEM((CHUNK, B), jnp.float32),
            pltpu.SemaphoreType.DMA,
            pltpu.SemaphoreType.DMA,
            pltpu.SemaphoreType.DMA,
            pltpu.SemaphoreType.DMA,
        ),
    )
    def k(x_hbm, src_hbm, nbr_hbm, attr_hbm, nbr_out, e_out, deg_out,
          acc_nbr, acc_e, acc_deg, sidx, nidx, rows, attr, ones,
          gsem0, gsem1, asem0, asem1):
        gsems = (gsem0, gsem1)
        asems = (asem0, asem1)
        c = lax.axis_index("c")
        s = lax.axis_index("s")
        ebase = (c * NS + s) * EPT

        zv = jnp.zeros((16,), jnp.float32)
        ov = jnp.ones((16,), jnp.float32)

        @pl.loop(0, 2 * CHUNK)
        def _(r):
            for j in range(F // 16):
                rows[r, pl.ds(j * 16, 16)] = zv
            attr[r, :] = zv

        @pl.loop(0, CHUNK)
        def _(r):
            ones[r, :] = ov

        def zero_range(base, nrows):
            done = 0
            while done < nrows:
                n = min(ZROWS, nrows - done)
                sub = pl.ds(base + done, n)
                zsrc = pl.ds(0, n)
                pltpu.sync_copy(rows.at[zsrc], acc_nbr.at[sub])
                pltpu.sync_copy(attr.at[zsrc], acc_e.at[sub])
                pltpu.sync_copy(attr.at[zsrc], acc_deg.at[sub])
                done += n

        zero_range(s * WRPT, WRPT)

        @pl.when(s == 0)
        def _():
            zsub = pl.ds(NS * WRPT, N_NODES - NS * WRPT)
            zsrc = pl.ds(0, N_NODES - NS * WRPT)
            pltpu.sync_copy(rows.at[zsrc], acc_nbr.at[zsub])
            pltpu.sync_copy(attr.at[zsrc], acc_e.at[zsub])
            pltpu.sync_copy(attr.at[zsrc], acc_deg.at[zsub])
        plsc.subcore_barrier()

        def rows_buf(b):
            return rows.at[pl.ds(b * CHUNK, CHUNK)]

        def attr_buf(b):
            return attr.at[pl.ds(b * CHUNK, CHUNK)]

        def load_idx(i, b):
            off = ebase + i * CHUNK
            pltpu.sync_copy(src_hbm.at[pl.ds(off, CHUNK)], sidx.at[b])
            pltpu.sync_copy(nbr_hbm.at[pl.ds(off, CHUNK)], nidx.at[b])

        def start_fetch(i, b):
            off = ebase + i * CHUNK
            pltpu.async_copy(x_hbm.at[nidx.at[b]], rows_buf(b), gsems[b])
            pltpu.async_copy(attr_hbm.at[pl.ds(off, CHUNK)], attr_buf(b),
                             asems[b])

        def process(j, b):
            off = ebase + j * CHUNK
            pltpu.make_async_copy(x_hbm.at[nidx.at[b]], rows_buf(b),
                                  gsems[b]).wait()
            pltpu.make_async_copy(attr_hbm.at[pl.ds(off, CHUNK)],
                                  attr_buf(b), asems[b]).wait()
            pltpu.sync_copy(rows_buf(b), acc_nbr.at[sidx.at[b]], add=True)
            pltpu.sync_copy(attr_buf(b), acc_e.at[sidx.at[b]], add=True)
            pltpu.sync_copy(ones, acc_deg.at[sidx.at[b]], add=True)

            @pl.when(j + 2 < NCHUNK)
            def _():
                load_idx(j + 2, b)
                start_fetch(j + 2, b)

        load_idx(0, 0)
        start_fetch(0, 0)
        load_idx(1, 1)
        start_fetch(1, 1)

        @pl.loop(0, NCHUNK, step=2)
        def _(i):
            for b in range(2):
                process(i + b, b)

        plsc.subcore_barrier()

        wsub = pl.ds(s * WRPT, WRPT)
        pltpu.sync_copy(acc_nbr.at[wsub], nbr_out.at[c, wsub])
        pltpu.sync_copy(acc_e.at[wsub], e_out.at[c, wsub])
        pltpu.sync_copy(acc_deg.at[wsub], deg_out.at[c, wsub])

        @pl.when(s == 0)
        def _():
            rsub = pl.ds(NS * WRPT, N_NODES - NS * WRPT)
            pltpu.sync_copy(acc_nbr.at[rsub], nbr_out.at[c, rsub])
            pltpu.sync_copy(acc_e.at[rsub], e_out.at[c, rsub])
            pltpu.sync_copy(acc_deg.at[rsub], deg_out.at[c, rsub])

    return k(x, src, nbr, edge_attr)


def _tc_combine(x, nbr_part, e_part, deg_part, w_s, w_nx, w_ne):
    R = 1000

    def body(x_ref, nbr_ref, e_ref, deg_ref, ws_ref, wnx_ref, wne_ref,
             out_ref):
        xv = x_ref[...]
        deg = deg_ref[0, :, 0:1] + deg_ref[1, :, 0:1]
        y = deg * xv + nbr_ref[0] + nbr_ref[1]
        e = e_ref[0] + e_ref[1]
        acc = jnp.dot(xv, ws_ref[...], preferred_element_type=jnp.float32)
        acc = acc + jnp.dot(y, wnx_ref[...],
                            preferred_element_type=jnp.float32)
        acc = acc + jnp.dot(e, wne_ref[...],
                            preferred_element_type=jnp.float32)
        out_ref[...] = acc

    return pl.pallas_call(
        body,
        grid=(N_NODES // R,),
        in_specs=[
            pl.BlockSpec((R, F), lambda i: (i, 0)),
            pl.BlockSpec((NC, R, F), lambda i: (0, i, 0)),
            pl.BlockSpec((NC, R, B), lambda i: (0, i, 0)),
            pl.BlockSpec((NC, R, B), lambda i: (0, i, 0)),
            pl.BlockSpec((F, F), lambda i: (0, 0)),
            pl.BlockSpec((F, F), lambda i: (0, 0)),
            pl.BlockSpec((B, F), lambda i: (0, 0)),
        ],
        out_specs=pl.BlockSpec((R, F), lambda i: (i, 0)),
        out_shape=jax.ShapeDtypeStruct((N_NODES, F), jnp.float32),
    )(x, nbr_part, e_part, deg_part, w_s, w_nx, w_ne)


def kernel(x, edge_index, edge_attr, w_s, w_n):
    src = edge_index[0]
    nbr = edge_index[1]
    w_nx = w_n[:F]
    w_ne = w_n[F:]
    nbr_part, e_part, deg_part = _sc_segment_sums(x, src, nbr, edge_attr)
    return _tc_combine(x, nbr_part, e_part, deg_part, w_s, w_nx, w_ne)


# speedup vs baseline: 5.5648x; 1.0338x over previous
import functools

import jax
import jax.numpy as jnp
from jax import lax
from jax.experimental import pallas as pl
from jax.experimental.pallas import tpu as pltpu
from jax.experimental.pallas import tpu_sc as plsc

N_NODES = 10000
N_EDGES = 320000
F = 128
B = 16
BA = 24
NC = 2
NS = 16
EPT = N_EDGES // (NC * NS)
CHUNK = 40
NCHUNK = EPT // CHUNK
BLK = 16
NBLK_FULL = NCHUNK // BLK
BLK_REM = NCHUNK - NBLK_FULL * BLK
WRPT = 624


def _sc_segment_sums(x, ei3, attr_aug, zeros_ba):
    mesh = plsc.VectorSubcoreMesh(core_axis_name="c", subcore_axis_name="s")

    @functools.partial(
        pl.kernel,
        out_type=(
            jax.ShapeDtypeStruct((NC, N_NODES, F), jnp.float32),
            jax.ShapeDtypeStruct((NC, N_NODES, BA), jnp.float32),
        ),
        mesh=mesh,
        compiler_params=pltpu.CompilerParams(use_tc_tiling_on_sc=False),
        scratch_types=(
            pltpu.VMEM_SHARED((N_NODES, F), jnp.float32),
            pltpu.VMEM_SHARED((N_NODES, BA), jnp.float32),
            pltpu.VMEM((2, BLK, CHUNK), jnp.int32),
            pltpu.VMEM((2, BLK, CHUNK), jnp.int32),
            pltpu.VMEM((2, CHUNK, F), jnp.float32),
            pltpu.VMEM((2, CHUNK, BA), jnp.float32),
            pltpu.SemaphoreType.DMA,
            pltpu.SemaphoreType.DMA,
            pltpu.SemaphoreType.DMA,
            pltpu.SemaphoreType.DMA,
        ),
    )
    def k(x_hbm, ei_hbm, attr_hbm, z_hbm, nbr_out, ed_out,
          acc_nbr, acc_ed, sblk, nblk, rows, attr,
          gsem0, gsem1, asem0, asem1):
        gsems = (gsem0, gsem1)
        asems = (asem0, asem1)
        c = lax.axis_index("c")
        s = lax.axis_index("s")
        t = c * NS + s
        rbase = t * NCHUNK

        zv = jnp.zeros((16,), jnp.float32)

        @pl.loop(0, CHUNK)
        def _(r):
            for j in range(F // 16):
                rows[0, r, pl.ds(j * 16, 16)] = zv

        done = 0
        while done < WRPT:
            n = min(CHUNK, WRPT - done)
            pltpu.sync_copy(rows.at[0, pl.ds(0, n)],
                            acc_nbr.at[pl.ds(s * WRPT + done, n)])
            done += n
        pltpu.sync_copy(z_hbm.at[pl.ds(s * WRPT, WRPT)],
                        acc_ed.at[pl.ds(s * WRPT, WRPT)])

        @pl.when(s == 0)
        def _():
            rem = N_NODES - NS * WRPT
            pltpu.sync_copy(rows.at[0, pl.ds(0, rem)],
                            acc_nbr.at[pl.ds(NS * WRPT, rem)])
            pltpu.sync_copy(z_hbm.at[pl.ds(NS * WRPT, rem)],
                            acc_ed.at[pl.ds(NS * WRPT, rem)])
        plsc.subcore_barrier()

        def load_blk(n, nrows, buf):
            pltpu.sync_copy(ei_hbm.at[0, pl.ds(rbase + n * BLK, nrows)],
                            sblk.at[buf, pl.ds(0, nrows)])
            pltpu.sync_copy(ei_hbm.at[1, pl.ds(rbase + n * BLK, nrows)],
                            nblk.at[buf, pl.ds(0, nrows)])

        def sidx(j):
            return sblk.at[lax.rem(j // BLK, 2), lax.rem(j, BLK)]

        def nidx(j):
            return nblk.at[lax.rem(j // BLK, 2), lax.rem(j, BLK)]

        def start_fetch(j, b):
            off = (rbase + j) * CHUNK
            pltpu.async_copy(x_hbm.at[nidx(j)], rows.at[b], gsems[b])
            pltpu.async_copy(attr_hbm.at[pl.ds(off, CHUNK)], attr.at[b],
                             asems[b])

        def process(j, b):
            off = (rbase + j) * CHUNK
            pltpu.make_async_copy(x_hbm.at[nidx(j)], rows.at[b],
                                  gsems[b]).wait()
            pltpu.make_async_copy(attr_hbm.at[pl.ds(off, CHUNK)],
                                  attr.at[b], asems[b]).wait()
            pltpu.sync_copy(rows.at[b], acc_nbr.at[sidx(j)], add=True)
            pltpu.sync_copy(attr.at[b], acc_ed.at[sidx(j)], add=True)

            @pl.when(lax.rem(j, BLK) == 0)
            def _():
                nxt = j // BLK + 1

                @pl.when(nxt < NBLK_FULL)
                def _():
                    load_blk(nxt, BLK, lax.rem(nxt, 2))

                @pl.when(nxt == NBLK_FULL)
                def _():
                    load_blk(nxt, BLK_REM, lax.rem(nxt, 2))

            @pl.when(j + 2 < NCHUNK)
            def _():
                start_fetch(j + 2, b)

        load_blk(0, BLK, 0)
        start_fetch(0, 0)
        start_fetch(1, 1)

        @pl.loop(0, NCHUNK, step=2)
        def _(i):
            for b in range(2):
                process(i + b, b)

        plsc.subcore_barrier()

        wsub = pl.ds(s * WRPT, WRPT)
        pltpu.sync_copy(acc_nbr.at[wsub], nbr_out.at[c, wsub])
        pltpu.sync_copy(acc_ed.at[wsub], ed_out.at[c, wsub])

        @pl.when(s == 0)
        def _():
            rsub = pl.ds(NS * WRPT, N_NODES - NS * WRPT)
            pltpu.sync_copy(acc_nbr.at[rsub], nbr_out.at[c, rsub])
            pltpu.sync_copy(acc_ed.at[rsub], ed_out.at[c, rsub])

    return k(x, ei3, attr_aug, zeros_ba)


def _tc_combine(x, nbr_part, ed_part, w_s, w_nx, w_ne):
    R = 1000

    def body(x_ref, nbr_ref, ed_ref, ws_ref, wnx_ref, wne_ref, out_ref):
        xv = x_ref[...]
        deg = ed_ref[0, :, B:B + 1] + ed_ref[1, :, B:B + 1]
        y = deg * xv + nbr_ref[0] + nbr_ref[1]
        e = ed_ref[0, :, :B] + ed_ref[1, :, :B]
        acc = jnp.dot(xv, ws_ref[...], preferred_element_type=jnp.float32)
        acc = acc + jnp.dot(y, wnx_ref[...],
                            preferred_element_type=jnp.float32)
        acc = acc + jnp.dot(e, wne_ref[...],
                            preferred_element_type=jnp.float32)
        out_ref[...] = acc

    return pl.pallas_call(
        body,
        grid=(N_NODES // R,),
        in_specs=[
            pl.BlockSpec((R, F), lambda i: (i, 0)),
            pl.BlockSpec((NC, R, F), lambda i: (0, i, 0)),
            pl.BlockSpec((NC, R, BA), lambda i: (0, i, 0)),
            pl.BlockSpec((F, F), lambda i: (0, 0)),
            pl.BlockSpec((F, F), lambda i: (0, 0)),
            pl.BlockSpec((B, F), lambda i: (0, 0)),
        ],
        out_specs=pl.BlockSpec((R, F), lambda i: (i, 0)),
        out_shape=jax.ShapeDtypeStruct((N_NODES, F), jnp.float32),
    )(x, nbr_part, ed_part, w_s, w_nx, w_ne)


def kernel(x, edge_index, edge_attr, w_s, w_n):
    ei3 = edge_index.reshape(2, N_EDGES // CHUNK, CHUNK)
    attr_aug = jnp.concatenate(
        [edge_attr, jnp.ones((N_EDGES, BA - B), jnp.float32)], axis=1)
    zeros_ba = jnp.zeros((N_NODES, BA), jnp.float32)
    w_nx = w_n[:F]
    w_ne = w_n[F:]
    nbr_part, ed_part = _sc_segment_sums(x, ei3, attr_aug, zeros_ba)
    return _tc_combine(x, nbr_part, ed_part, w_s, w_nx, w_ne)


# speedup vs baseline: 7.5206x; 1.3514x over previous
import functools

import jax
import jax.numpy as jnp
from jax import lax
from jax.experimental import pallas as pl
from jax.experimental.pallas import tpu as pltpu
from jax.experimental.pallas import tpu_sc as plsc

N_NODES = 10000
N_EDGES = 320000
F = 128
B = 16
BA = 24
NC = 2
NS = 16
EPT = N_EDGES // (NC * NS)
CHUNK = 40
NCHUNK = EPT // CHUNK
BLK = 16
BLKE = BLK * CHUNK
NBLK_FULL = NCHUNK // BLK
BLK_REM = NCHUNK - NBLK_FULL * BLK
WRPT = 624


def _sc_segment_sums(x, edge_index, edge_attr, ones_pad, zeros_ba):
    mesh = plsc.VectorSubcoreMesh(core_axis_name="c", subcore_axis_name="s")

    @functools.partial(
        pl.kernel,
        out_type=(
            jax.ShapeDtypeStruct((NC, N_NODES, F), jnp.float32),
            jax.ShapeDtypeStruct((NC, N_NODES, BA), jnp.float32),
        ),
        mesh=mesh,
        compiler_params=pltpu.CompilerParams(use_tc_tiling_on_sc=False),
        scratch_types=(
            pltpu.VMEM_SHARED((N_NODES, F), jnp.float32),
            pltpu.VMEM_SHARED((N_NODES, BA), jnp.float32),
            pltpu.VMEM((2, BLKE), jnp.int32),
            pltpu.VMEM((2, BLKE), jnp.int32),
            pltpu.VMEM((2, CHUNK, F), jnp.float32),
            pltpu.VMEM((2, CHUNK, BA), jnp.float32),
            pltpu.SemaphoreType.DMA,
            pltpu.SemaphoreType.DMA,
            pltpu.SemaphoreType.DMA,
            pltpu.SemaphoreType.DMA,
        ),
    )
    def k(x_hbm, ei_hbm, attr_hbm, op_hbm, z_hbm, nbr_out, ed_out,
          acc_nbr, acc_ed, sblk, nblk, rows, attr,
          gsem0, gsem1, asem0, asem1):
        gsems = (gsem0, gsem1)
        asems = (asem0, asem1)
        c = lax.axis_index("c")
        s = lax.axis_index("s")
        t = c * NS + s
        ebase = t * EPT

        zv = jnp.zeros((16,), jnp.float32)

        @pl.loop(0, CHUNK)
        def _(r):
            for j in range(F // 16):
                rows[0, r, pl.ds(j * 16, 16)] = zv

        done = 0
        while done < WRPT:
            n = min(CHUNK, WRPT - done)
            pltpu.sync_copy(rows.at[0, pl.ds(0, n)],
                            acc_nbr.at[pl.ds(s * WRPT + done, n)])
            done += n
        pltpu.sync_copy(z_hbm.at[pl.ds(s * WRPT, WRPT)],
                        acc_ed.at[pl.ds(s * WRPT, WRPT)])

        @pl.when(s == 0)
        def _():
            rem = N_NODES - NS * WRPT
            pltpu.sync_copy(rows.at[0, pl.ds(0, rem)],
                            acc_nbr.at[pl.ds(NS * WRPT, rem)])
            pltpu.sync_copy(z_hbm.at[pl.ds(NS * WRPT, rem)],
                            acc_ed.at[pl.ds(NS * WRPT, rem)])

        for b in range(2):
            pltpu.sync_copy(op_hbm, attr.at[b, :, pl.ds(B, BA - B)])
        plsc.subcore_barrier()

        def load_blk(n, nedges, buf):
            off = ebase + n * BLKE
            pltpu.sync_copy(ei_hbm.at[0, pl.ds(off, nedges)],
                            sblk.at[buf, pl.ds(0, nedges)])
            pltpu.sync_copy(ei_hbm.at[1, pl.ds(off, nedges)],
                            nblk.at[buf, pl.ds(0, nedges)])

        def sidx(j):
            return sblk.at[lax.rem(j // BLK, 2),
                           pl.ds(lax.rem(j, BLK) * CHUNK, CHUNK)]

        def nidx(j):
            return nblk.at[lax.rem(j // BLK, 2),
                           pl.ds(lax.rem(j, BLK) * CHUNK, CHUNK)]

        def start_fetch(j, b):
            off = ebase + j * CHUNK
            pltpu.async_copy(x_hbm.at[nidx(j)], rows.at[b], gsems[b])
            pltpu.async_copy(attr_hbm.at[pl.ds(off, CHUNK)],
                             attr.at[b, :, pl.ds(0, B)], asems[b])

        def process(j, b):
            off = ebase + j * CHUNK
            pltpu.make_async_copy(x_hbm.at[nidx(j)], rows.at[b],
                                  gsems[b]).wait()
            pltpu.make_async_copy(attr_hbm.at[pl.ds(off, CHUNK)],
                                  attr.at[b, :, pl.ds(0, B)],
                                  asems[b]).wait()
            pltpu.sync_copy(rows.at[b], acc_nbr.at[sidx(j)], add=True)
            pltpu.sync_copy(attr.at[b], acc_ed.at[sidx(j)], add=True)

            @pl.when(lax.rem(j, BLK) == 0)
            def _():
                nxt = j // BLK + 1

                @pl.when(nxt < NBLK_FULL)
                def _():
                    load_blk(nxt, BLKE, lax.rem(nxt, 2))

                @pl.when(nxt == NBLK_FULL)
                def _():
                    load_blk(nxt, BLK_REM * CHUNK, lax.rem(nxt, 2))

            @pl.when(j + 2 < NCHUNK)
            def _():
                start_fetch(j + 2, b)

        load_blk(0, BLKE, 0)
        start_fetch(0, 0)
        start_fetch(1, 1)

        @pl.loop(0, NCHUNK, step=2)
        def _(i):
            for b in range(2):
                process(i + b, b)

        plsc.subcore_barrier()

        wsub = pl.ds(s * WRPT, WRPT)
        pltpu.sync_copy(acc_nbr.at[wsub], nbr_out.at[c, wsub])
        pltpu.sync_copy(acc_ed.at[wsub], ed_out.at[c, wsub])

        @pl.when(s == 0)
        def _():
            rsub = pl.ds(NS * WRPT, N_NODES - NS * WRPT)
            pltpu.sync_copy(acc_nbr.at[rsub], nbr_out.at[c, rsub])
            pltpu.sync_copy(acc_ed.at[rsub], ed_out.at[c, rsub])

    return k(x, edge_index, edge_attr, ones_pad, zeros_ba)


def _tc_combine(x, nbr_part, ed_part, w_s, w_nx, w_ne):
    R = 1000

    def body(x_ref, nbr_ref, ed_ref, ws_ref, wnx_ref, wne_ref, out_ref):
        xv = x_ref[...]
        deg = ed_ref[0, :, B:B + 1] + ed_ref[1, :, B:B + 1]
        y = deg * xv + nbr_ref[0] + nbr_ref[1]
        e = ed_ref[0, :, :B] + ed_ref[1, :, :B]
        acc = jnp.dot(xv, ws_ref[...], preferred_element_type=jnp.float32)
        acc = acc + jnp.dot(y, wnx_ref[...],
                            preferred_element_type=jnp.float32)
        acc = acc + jnp.dot(e, wne_ref[...],
                            preferred_element_type=jnp.float32)
        out_ref[...] = acc

    return pl.pallas_call(
        body,
        grid=(N_NODES // R,),
        in_specs=[
            pl.BlockSpec((R, F), lambda i: (i, 0)),
            pl.BlockSpec((NC, R, F), lambda i: (0, i, 0)),
            pl.BlockSpec((NC, R, BA), lambda i: (0, i, 0)),
            pl.BlockSpec((F, F), lambda i: (0, 0)),
            pl.BlockSpec((F, F), lambda i: (0, 0)),
            pl.BlockSpec((B, F), lambda i: (0, 0)),
        ],
        out_specs=pl.BlockSpec((R, F), lambda i: (i, 0)),
        out_shape=jax.ShapeDtypeStruct((N_NODES, F), jnp.float32),
    )(x, nbr_part, ed_part, w_s, w_nx, w_ne)


def kernel(x, edge_index, edge_attr, w_s, w_n):
    ones_pad = jnp.ones((CHUNK, BA - B), jnp.float32)
    zeros_ba = jnp.zeros((N_NODES, BA), jnp.float32)
    w_nx = w_n[:F]
    w_ne = w_n[F:]
    nbr_part, ed_part = _sc_segment_sums(x, edge_index, edge_attr, ones_pad,
                                         zeros_ba)
    return _tc_combine(x, nbr_part, ed_part, w_s, w_nx, w_ne)
